# baseline (speedup 1.0000x reference)
import functools

import jax
import jax.numpy as jnp
from jax import lax
from jax.experimental import pallas as pl
from jax.experimental.pallas import tpu as pltpu
from jax.experimental.pallas import tpu_sc as plsc

_N = 10000
_E = 320000
_TIN = 6
_STEPS = 3
_D = 64
_NB = 2000
_EB = 3200
_NW = 32
_PERW = _E // _NW
_CHUNK = 200
_NCHUNKS = _PERW // _CHUNK
_HALFP = 5120

_F32 = jnp.float32


def _vec(b):
    return b.reshape(1, -1)


def _full(a):
    return pl.BlockSpec(a.shape, lambda i: (0,) * a.ndim)


def _node_prep(nodein, w1v, te9, gconst, w2n, b2n, w1s, w1d):
    nb = _NB

    def body(nin_ref, w1v_ref, te_ref, gc_ref, w2_ref, b2_ref, ws_ref, wd_ref,
             h_ref, ts_ref, td_ref):
        x = nin_ref[...]
        vel = x[:, 3:18] - x[:, 0:15]
        pt = x[:, 18:19]
        tids = lax.broadcasted_iota(jnp.int32, (1, 9), 1).astype(_F32)
        oh = (pt == tids).astype(_F32)
        z = jnp.dot(vel, w1v_ref[...], preferred_element_type=_F32)
        z = z + jnp.dot(oh, te_ref[...], preferred_element_type=_F32)
        z = jax.nn.relu(z + gc_ref[...])
        h = jnp.dot(z, w2_ref[...], preferred_element_type=_F32) + b2_ref[...]
        h_ref[...] = h
        last = x[:, 15:18]
        ps = jnp.dot(h, ws_ref[...], preferred_element_type=_F32)
        pd = jnp.dot(h, wd_ref[...], preferred_element_type=_F32)
        par = (lax.broadcasted_iota(jnp.int32, (nb, 1), 0) % 2).astype(_F32)
        pad = jnp.zeros((nb, 60), _F32)
        zcol = jnp.zeros((nb, 1), _F32)
        ts_ref[...] = jnp.concatenate([ps, last, zcol, pad], axis=1)
        td_ref[...] = jnp.concatenate([pd, -last, par, pad], axis=1)

    return pl.pallas_call(
        body,
        grid=(_N // nb,),
        in_specs=[pl.BlockSpec((nb, 19), lambda i: (i, 0)),
                  _full(w1v), _full(te9), _full(gconst), _full(w2n),
                  _full(b2n), _full(w1s), _full(w1d)],
        out_specs=[pl.BlockSpec((nb, _D), lambda i: (i, 0)),
                   pl.BlockSpec((nb, 128), lambda i: (i, 0)),
                   pl.BlockSpec((nb, 128), lambda i: (i, 0))],
        out_shape=[jax.ShapeDtypeStruct((_N, _D), _F32),
                   jax.ShapeDtypeStruct((_N, 128), _F32),
                   jax.ShapeDtypeStruct((_N, 128), _F32)],
    )(nodein, w1v, te9, gconst, w2n, b2n, w1s, w1d)


def _edge_round1(g80, w1e, b1e, w2e, b2e, w1me, b1m, w2m, b2m):
    eb = _EB

    def body(g_ref, w1e_ref, b1e_ref, w2e_ref, b2e_ref, w1m_ref, b1m_ref,
             w2m_ref, b2m_ref, e1_ref, m_ref):
        g = g_ref[...]
        c = g[:, 0:64]
        rel = g[:, 64:67]
        dist = jnp.sqrt(jnp.sum(rel * rel, axis=1, keepdims=True) + 1e-12)
        r4 = jnp.concatenate([rel, dist], axis=1)
        z = jax.nn.relu(jnp.dot(r4, w1e_ref[...], preferred_element_type=_F32)
                        + b1e_ref[...])
        e0 = jnp.dot(z, w2e_ref[...], preferred_element_type=_F32) + b2e_ref[...]
        x = jax.nn.relu(c + jnp.dot(e0, w1m_ref[...],
                                    preferred_element_type=_F32)
                        + b1m_ref[...])
        m = jnp.dot(x, w2m_ref[...], preferred_element_type=_F32) + b2m_ref[...]
        par = g[:, 67:68]
        m_ref[...] = jnp.concatenate([m * (1.0 - par), m * par], axis=1)
        e1_ref[...] = e0 + m

    return pl.pallas_call(
        body,
        grid=(_E // eb,),
        in_specs=[pl.BlockSpec((eb, 128), lambda i: (i, 0)),
                  _full(w1e), _full(b1e), _full(w2e), _full(b2e),
                  _full(w1me), _full(b1m), _full(w2m), _full(b2m)],
        out_specs=[pl.BlockSpec((eb, _D), lambda i: (i, 0)),
                   pl.BlockSpec((eb, 128), lambda i: (i, 0))],
        out_shape=[jax.ShapeDtypeStruct((_E, _D), _F32),
                   jax.ShapeDtypeStruct((_E, 128), _F32)],
    )(g80, w1e, b1e, w2e, b2e, w1me, b1m, w2m, b2m)


def _edge_round2(e1, g64, w1me, b1m, w2m, b2m):
    eb = _EB

    def body(e_ref, g_ref, w1m_ref, b1m_ref, w2m_ref, b2m_ref, m_ref):
        x = jax.nn.relu(g_ref[:, 0:64]
                        + jnp.dot(e_ref[...], w1m_ref[...],
                                  preferred_element_type=_F32)
                        + b1m_ref[...])
        m = jnp.dot(x, w2m_ref[...], preferred_element_type=_F32) \
            + b2m_ref[...]
        par = g_ref[:, 64:65]
        m_ref[...] = jnp.concatenate([m * (1.0 - par), m * par], axis=1)

    return pl.pallas_call(
        body,
        grid=(_E // eb,),
        in_specs=[pl.BlockSpec((eb, _D), lambda i: (i, 0)),
                  pl.BlockSpec((eb, 128), lambda i: (i, 0)),
                  _full(w1me), _full(b1m), _full(w2m), _full(b2m)],
        out_specs=pl.BlockSpec((eb, 128), lambda i: (i, 0)),
        out_shape=jax.ShapeDtypeStruct((_E, 128), _F32),
    )(e1, g64, w1me, b1m, w2m, b2m)


def _node_update(h, a0, a1, w1h, w1a, b1, w2, b2, wns, wnd):
    nb = _NB

    def body(h_ref, a0_ref, a1_ref, w1h_ref, w1a_ref, b1_ref, w2_ref, b2_ref,
             ws_ref, wd_ref, hn_ref, ts_ref, td_ref):
        h_ = h_ref[...]
        agg = a0_ref[...] + a1_ref[...]
        z = jax.nn.relu(jnp.dot(h_, w1h_ref[...], preferred_element_type=_F32)
                        + jnp.dot(agg, w1a_ref[...], preferred_element_type=_F32)
                        + b1_ref[...])
        hn = h_ + jnp.dot(z, w2_ref[...], preferred_element_type=_F32) \
            + b2_ref[...]
        hn_ref[...] = hn
        nb_ = h_.shape[0]
        par = (lax.broadcasted_iota(jnp.int32, (nb_, 1), 0) % 2).astype(_F32)
        pad = jnp.zeros((nb_, 63), _F32)
        zcol = jnp.zeros((nb_, 1), _F32)
        ts_ref[...] = jnp.concatenate(
            [jnp.dot(hn, ws_ref[...], preferred_element_type=_F32), zcol, pad],
            axis=1)
        td_ref[...] = jnp.concatenate(
            [jnp.dot(hn, wd_ref[...], preferred_element_type=_F32), par, pad],
            axis=1)

    return pl.pallas_call(
        body,
        grid=(_N // nb,),
        in_specs=[pl.BlockSpec((nb, _D), lambda i: (i, 0)),
                  pl.BlockSpec((nb, _D), lambda i: (i, 0)),
                  pl.BlockSpec((nb, _D), lambda i: (i, 0)),
                  _full(w1h), _full(w1a), _full(b1), _full(w2), _full(b2),
                  _full(wns), _full(wnd)],
        out_specs=[pl.BlockSpec((nb, _D), lambda i: (i, 0)),
                   pl.BlockSpec((nb, 128), lambda i: (i, 0)),
                   pl.BlockSpec((nb, 128), lambda i: (i, 0))],
        out_shape=[jax.ShapeDtypeStruct((_N, _D), _F32),
                   jax.ShapeDtypeStruct((_N, 128), _F32),
                   jax.ShapeDtypeStruct((_N, 128), _F32)],
    )(h, a0, a1, w1h, w1a, b1, w2, b2, wns, wnd)


def _node_final(h, a0, a1, aux, w1h, w1a, b1, w2, b2, wd1, bd1, wd2, bd2):
    nb = _NB

    def body(h_ref, a0_ref, a1_ref, aux_ref, w1h_ref, w1a_ref, b1_ref,
             w2_ref, b2_ref, wd1_ref, bd1_ref, wd2_ref, bd2_ref,
             nxt_ref, loss_ref):
        h_ = h_ref[...]
        agg = a0_ref[...] + a1_ref[...]
        z = jax.nn.relu(jnp.dot(h_, w1h_ref[...], preferred_element_type=_F32)
                        + jnp.dot(agg, w1a_ref[...], preferred_element_type=_F32)
                        + b1_ref[...])
        hn = h_ + jnp.dot(z, w2_ref[...], preferred_element_type=_F32) \
            + b2_ref[...]
        za = jax.nn.relu(jnp.dot(hn, wd1_ref[...], preferred_element_type=_F32)
                         + bd1_ref[...])
        accel = jnp.dot(za, wd2_ref[...], preferred_element_type=_F32) \
            + bd2_ref[...]
        aux_ = aux_ref[...]
        last = aux_[:, 0:3]
        prev = aux_[:, 3:6]
        gt = aux_[:, 6:9]
        pt = aux_[:, 9:10]
        nxt = last + (last - prev) + accel
        nxt = jnp.where(pt == 3.0, gt, nxt)
        nxt_ref[...] = nxt
        df = nxt - gt

        @pl.when(pl.program_id(0) == 0)
        def _():
            loss_ref[0, 0] = 0.0

        loss_ref[0, 0] += jnp.sum(df * df)

    return pl.pallas_call(
        body,
        grid=(_N // nb,),
        in_specs=[pl.BlockSpec((nb, _D), lambda i: (i, 0)),
                  pl.BlockSpec((nb, _D), lambda i: (i, 0)),
                  pl.BlockSpec((nb, _D), lambda i: (i, 0)),
                  pl.BlockSpec((nb, 10), lambda i: (i, 0)),
                  _full(w1h), _full(w1a), _full(b1), _full(w2), _full(b2),
                  _full(wd1), _full(bd1), _full(wd2), _full(bd2)],
        out_specs=[pl.BlockSpec((nb, 3), lambda i: (i, 0)),
                   pl.BlockSpec((1, 1), lambda i: (0, 0),
                                memory_space=pltpu.SMEM)],
        out_shape=[jax.ShapeDtypeStruct((_N, 3), _F32),
                   jax.ShapeDtypeStruct((1, 1), _F32)],
    )(h, a0, a1, aux, w1h, w1a, b1, w2, b2, wd1, bd1, wd2, bd2)


def _make_gather(width):
    mesh = plsc.VectorSubcoreMesh(core_axis_name="c", subcore_axis_name="s")

    @functools.partial(
        pl.kernel,
        out_type=jax.ShapeDtypeStruct((_E, width), _F32),
        mesh=mesh,
        scratch_types=[
            pltpu.VMEM((_CHUNK,), jnp.int32),
            pltpu.VMEM((_CHUNK,), jnp.int32),
            pltpu.VMEM((_CHUNK,), jnp.int32),
            pltpu.VMEM((_CHUNK,), jnp.int32),
            pltpu.VMEM((_CHUNK, width), _F32),
            pltpu.VMEM((_CHUNK, width), _F32),
            pltpu.VMEM((_CHUNK, width), _F32),
            pltpu.VMEM((_CHUNK, width), _F32),
            pltpu.SemaphoreType.DMA,
            pltpu.SemaphoreType.DMA,
            pltpu.SemaphoreType.DMA,
            pltpu.SemaphoreType.DMA,
        ],
    )
    def gather_kernel(ts, td, src, dst, out, sidx0, didx0, sidx1, didx1,
                      bufa0, bufb0, bufa1, bufb1, sa0, sb0, sa1, sb1):
        cid = lax.axis_index("c")
        sid = lax.axis_index("s")
        base = (cid * 16 + sid) * _PERW
        slots = ((sidx0, didx0, bufa0, bufb0, sa0, sb0),
                 (sidx1, didx1, bufa1, bufb1, sa1, sb1))

        def issue(j, b):
            off = base + j * _CHUNK
            si, di, ba, bb, sas, sbs = slots[b]
            pltpu.sync_copy(src.at[pl.ds(off, _CHUNK)], si)
            pltpu.sync_copy(dst.at[pl.ds(off, _CHUNK)], di)
            pltpu.async_copy(ts.at[si], ba, sas)
            pltpu.async_copy(td.at[di], bb, sbs)

        def drain_and_emit(j, b):
            off = base + j * _CHUNK
            si, di, ba, bb, sas, sbs = slots[b]
            pltpu.make_async_copy(ts.at[si], ba, sas).wait()
            pltpu.make_async_copy(td.at[di], bb, sbs).wait()

            def row_body(i, carry2):
                for w in range(width // 16):
                    sl = pl.ds(w * 16, 16)
                    ba[i, sl] = ba[i, sl] + bb[i, sl]
                return carry2

            lax.fori_loop(0, _CHUNK, row_body, 0, unroll=4)
            pltpu.sync_copy(ba, out.at[pl.ds(off, _CHUNK)])

        issue(0, 0)

        def pair_body(j2, carry):
            j = j2 * 2
            issue(j + 1, 1)
            drain_and_emit(j, 0)

            @pl.when(j + 2 < _NCHUNKS)
            def _():
                issue(j + 2, 0)

            drain_and_emit(j + 1, 1)
            return carry

        lax.fori_loop(0, _NCHUNKS // 2, pair_body, 0)

    return gather_kernel


def _make_scatter():
    mesh = plsc.VectorSubcoreMesh(core_axis_name="c", subcore_axis_name="s")
    half = _HALFP
    rpt = half // 10

    @functools.partial(
        pl.kernel,
        out_type=jax.ShapeDtypeStruct((2 * half, 128), _F32),
        mesh=mesh,
        scratch_types=[
            pltpu.VMEM((_CHUNK,), jnp.int32),
            pltpu.VMEM((_CHUNK,), jnp.int32),
            pltpu.VMEM((_CHUNK, 128), _F32),
            pltpu.VMEM((_CHUNK, 128), _F32),
            pltpu.VMEM_SHARED((half, 128), _F32),
            pltpu.SemaphoreType.DMA,
            pltpu.SemaphoreType.DMA,
        ],
    )
    def scatter_kernel(m, dstv, zeros, out, didx0, didx1, mbuf0, mbuf1,
                       shared, sm0, sm1):
        cid = lax.axis_index("c")
        sid = lax.axis_index("s")

        @pl.when(sid == 0)
        def _():
            pltpu.sync_copy(zeros, shared)

        plsc.subcore_barrier()
        base = (cid * 16 + sid) * _PERW
        slots = ((didx0, mbuf0, sm0), (didx1, mbuf1, sm1))

        def issue(j, b):
            off = base + j * _CHUNK
            di, mb, sm = slots[b]
            pltpu.sync_copy(dstv.at[pl.ds(off, _CHUNK)], di)
            pltpu.async_copy(m.at[pl.ds(off, _CHUNK)], mb, sm)

        def drain_and_add(j, b):
            off = base + j * _CHUNK
            di, mb, sm = slots[b]
            pltpu.make_async_copy(m.at[pl.ds(off, _CHUNK)], mb, sm).wait()
            pltpu.sync_copy(mb, shared.at[di], add=True)

        issue(0, 0)

        def pair_body(j2, carry):
            j = j2 * 2
            issue(j + 1, 1)
            drain_and_add(j, 0)

            @pl.when(j + 2 < _NCHUNKS)
            def _():
                issue(j + 2, 0)

            drain_and_add(j + 1, 1)
            return carry

        lax.fori_loop(0, _NCHUNKS // 2, pair_body, 0)
        plsc.subcore_barrier()

        @pl.when(sid < 10)
        def _():
            r0 = sid * rpt
            pltpu.sync_copy(shared.at[pl.ds(r0, rpt)],
                            out.at[pl.ds(cid * half + r0, rpt)])

    return scatter_kernel


_gather128 = _make_gather(128)
_scatter = _make_scatter()


def kernel(position, n_particles_per_example, particle_type, step_context,
           edge_index, ct0, ct1, ct2, type_embedding, node_enc, edge_enc,
           mp1_edge, mp1_node, mp2_edge, mp2_node, decoder):
    del n_particles_per_example, step_context
    (w1n, b1n), (w2n, b2n) = node_enc
    (w1e, b1e), (w2e, b2e) = edge_enc
    (w1m1, b1m1), (w2m1, b2m1) = mp1_edge
    (w1u1, b1u1), (w2u1, b2u1) = mp1_node
    (w1m2, b1m2), (w2m2, b2m2) = mp2_edge
    (w1u2, b1u2), (w2u2, b2u2) = mp2_node
    (wd1, bd1), (wd2, bd2) = decoder

    gctx = jnp.concatenate([(ct0 * 2900.0 + 100.0).reshape(1),
                            (ct1 * 195.0 + 5.0).reshape(1),
                            (ct2 * 0.45).reshape(1)])
    te9 = type_embedding @ w1n[15:31]
    gconst = (gctx @ w1n[31:34] + b1n).reshape(1, _D)
    w1nv = w1n[0:15]

    ptype_f = particle_type.astype(_F32)[:, None]
    src = edge_index[0]
    dst = edge_index[1]
    dhalf = dst // 2
    cur = position[:, 0:_TIN]
    gt = position[:, _TIN:_TIN + _STEPS]
    zeros = jnp.zeros((_HALFP, 128), _F32)

    preds = []
    losses = []
    for step in range(_STEPS):
        nodein = jnp.concatenate([cur.reshape(_N, 18), ptype_f], axis=1)
        h, ts, td = _node_prep(nodein, w1nv, te9, gconst, w2n, _vec(b2n),
                               w1m1[64:128], w1m1[128:192])
        g80 = _gather128(ts, td, src, dst)
        e1, m1 = _edge_round1(g80, w1e, _vec(b1e), w2e, _vec(b2e),
                              w1m1[0:64], _vec(b1m1), w2m1, _vec(b2m1))
        agg1 = _scatter(m1, dhalf, zeros)
        a0 = agg1[:_N // 2].reshape(_N, _D)
        a1 = agg1[_HALFP:_HALFP + _N // 2].reshape(_N, _D)
        h, ts2, td2 = _node_update(h, a0, a1, w1u1[0:64], w1u1[64:128],
                                   _vec(b1u1), w2u1, _vec(b2u1),
                                   w1m2[64:128], w1m2[128:192])
        g64 = _gather128(ts2, td2, src, dst)
        m2 = _edge_round2(e1, g64, w1m2[0:64], _vec(b1m2), w2m2, _vec(b2m2))
        agg2 = _scatter(m2, dhalf, zeros)
        b0 = agg2[:_N // 2].reshape(_N, _D)
        b1_ = agg2[_HALFP:_HALFP + _N // 2].reshape(_N, _D)
        aux = jnp.concatenate([cur[:, -1], cur[:, -2], gt[:, step], ptype_f],
                              axis=1)
        nxt, lstep = _node_final(h, b0, b1_, aux, w1u2[0:64], w1u2[64:128],
                                 _vec(b1u2), w2u2, _vec(b2u2), wd1, _vec(bd1),
                                 wd2, _vec(bd2))
        preds.append(nxt)
        losses.append(lstep[0, 0])
        cur = jnp.concatenate([cur[:, 1:], nxt[:, None, :]], axis=1)

    preds = jnp.stack(preds)
    gt_p = jnp.transpose(gt, (1, 0, 2))
    loss = losses[0] + losses[1] + losses[2]
    return (loss, preds, gt_p)


# speedup vs baseline: 1.4432x; 1.4432x over previous
import functools

import jax
import jax.numpy as jnp
from jax import lax
from jax.experimental import pallas as pl
from jax.experimental.pallas import tpu as pltpu
from jax.experimental.pallas import tpu_sc as plsc

_N = 10000
_E = 320000
_TIN = 6
_STEPS = 3
_D = 64
_NB = 2000
_EB = 3200
_NW = 32
_PERW = _E // _NW
_CHUNK = 200
_NCHUNKS = _PERW // _CHUNK
_HALFP = 5120

_F32 = jnp.float32


def _vec(b):
    return b.reshape(1, -1)


def _full(a):
    return pl.BlockSpec(a.shape, lambda i: (0,) * a.ndim)


def _node_prep(nodein, w1v, te9, gconst, w2n, b2n, w1s, w1d):
    nb = _NB

    def body(nin_ref, w1v_ref, te_ref, gc_ref, w2_ref, b2_ref, ws_ref, wd_ref,
             h_ref, ts_ref, td_ref):
        x = nin_ref[...]
        vel = x[:, 3:18] - x[:, 0:15]
        pt = x[:, 18:19]
        tids = lax.broadcasted_iota(jnp.int32, (1, 9), 1).astype(_F32)
        oh = (pt == tids).astype(_F32)
        z = jnp.dot(vel, w1v_ref[...], preferred_element_type=_F32)
        z = z + jnp.dot(oh, te_ref[...], preferred_element_type=_F32)
        z = jax.nn.relu(z + gc_ref[...])
        h = jnp.dot(z, w2_ref[...], preferred_element_type=_F32) + b2_ref[...]
        h_ref[...] = h
        last = x[:, 15:18]
        ps = jnp.dot(h, ws_ref[...], preferred_element_type=_F32)
        pd = jnp.dot(h, wd_ref[...], preferred_element_type=_F32)
        par = (lax.broadcasted_iota(jnp.int32, (nb, 1), 0) % 2).astype(_F32)
        pad = jnp.zeros((nb, 60), _F32)
        zcol = jnp.zeros((nb, 1), _F32)
        ts_ref[...] = jnp.concatenate([ps, last, zcol, pad], axis=1)
        td_ref[...] = jnp.concatenate([pd, -last, par, pad], axis=1)

    return pl.pallas_call(
        body,
        grid=(_N // nb,),
        in_specs=[pl.BlockSpec((nb, 19), lambda i: (i, 0)),
                  _full(w1v), _full(te9), _full(gconst), _full(w2n),
                  _full(b2n), _full(w1s), _full(w1d)],
        out_specs=[pl.BlockSpec((nb, _D), lambda i: (i, 0)),
                   pl.BlockSpec((nb, 128), lambda i: (i, 0)),
                   pl.BlockSpec((nb, 128), lambda i: (i, 0))],
        out_shape=[jax.ShapeDtypeStruct((_N, _D), _F32),
                   jax.ShapeDtypeStruct((_N, 128), _F32),
                   jax.ShapeDtypeStruct((_N, 128), _F32)],
    )(nodein, w1v, te9, gconst, w2n, b2n, w1s, w1d)


def _edge_round1(g80, w1e, b1e, w2e, b2e, w1me, b1m, w2m, b2m):
    eb = _EB

    def body(g_ref, w1e_ref, b1e_ref, w2e_ref, b2e_ref, w1m_ref, b1m_ref,
             w2m_ref, b2m_ref, e1_ref, m_ref):
        g = g_ref[...]
        c = g[:, 0:64]
        rel = g[:, 64:67]
        dist = jnp.sqrt(jnp.sum(rel * rel, axis=1, keepdims=True) + 1e-12)
        r4 = jnp.concatenate([rel, dist], axis=1)
        z = jax.nn.relu(jnp.dot(r4, w1e_ref[...], preferred_element_type=_F32)
                        + b1e_ref[...])
        e0 = jnp.dot(z, w2e_ref[...], preferred_element_type=_F32) + b2e_ref[...]
        x = jax.nn.relu(c + jnp.dot(e0, w1m_ref[...],
                                    preferred_element_type=_F32)
                        + b1m_ref[...])
        m = jnp.dot(x, w2m_ref[...], preferred_element_type=_F32) + b2m_ref[...]
        par = g[:, 67:68]
        m_ref[...] = jnp.concatenate([m * (1.0 - par), m * par], axis=1)
        e1_ref[...] = e0 + m

    return pl.pallas_call(
        body,
        grid=(_E // eb,),
        in_specs=[pl.BlockSpec((eb, 128), lambda i: (i, 0)),
                  _full(w1e), _full(b1e), _full(w2e), _full(b2e),
                  _full(w1me), _full(b1m), _full(w2m), _full(b2m)],
        out_specs=[pl.BlockSpec((eb, _D), lambda i: (i, 0)),
                   pl.BlockSpec((eb, 128), lambda i: (i, 0))],
        out_shape=[jax.ShapeDtypeStruct((_E, _D), _F32),
                   jax.ShapeDtypeStruct((_E, 128), _F32)],
    )(g80, w1e, b1e, w2e, b2e, w1me, b1m, w2m, b2m)


def _edge_round2(e1, g64, w1me, b1m, w2m, b2m):
    eb = _EB

    def body(e_ref, g_ref, w1m_ref, b1m_ref, w2m_ref, b2m_ref, m_ref):
        x = jax.nn.relu(g_ref[:, 0:64]
                        + jnp.dot(e_ref[...], w1m_ref[...],
                                  preferred_element_type=_F32)
                        + b1m_ref[...])
        m = jnp.dot(x, w2m_ref[...], preferred_element_type=_F32) \
            + b2m_ref[...]
        par = g_ref[:, 64:65]
        m_ref[...] = jnp.concatenate([m * (1.0 - par), m * par], axis=1)

    return pl.pallas_call(
        body,
        grid=(_E // eb,),
        in_specs=[pl.BlockSpec((eb, _D), lambda i: (i, 0)),
                  pl.BlockSpec((eb, 128), lambda i: (i, 0)),
                  _full(w1me), _full(b1m), _full(w2m), _full(b2m)],
        out_specs=pl.BlockSpec((eb, 128), lambda i: (i, 0)),
        out_shape=jax.ShapeDtypeStruct((_E, 128), _F32),
    )(e1, g64, w1me, b1m, w2m, b2m)


def _node_update(h, a0, a1, w1h, w1a, b1, w2, b2, wns, wnd):
    nb = _NB

    def body(h_ref, a0_ref, a1_ref, w1h_ref, w1a_ref, b1_ref, w2_ref, b2_ref,
             ws_ref, wd_ref, hn_ref, ts_ref, td_ref):
        h_ = h_ref[...]
        agg = a0_ref[...] + a1_ref[...]
        z = jax.nn.relu(jnp.dot(h_, w1h_ref[...], preferred_element_type=_F32)
                        + jnp.dot(agg, w1a_ref[...], preferred_element_type=_F32)
                        + b1_ref[...])
        hn = h_ + jnp.dot(z, w2_ref[...], preferred_element_type=_F32) \
            + b2_ref[...]
        hn_ref[...] = hn
        nb_ = h_.shape[0]
        par = (lax.broadcasted_iota(jnp.int32, (nb_, 1), 0) % 2).astype(_F32)
        pad = jnp.zeros((nb_, 63), _F32)
        zcol = jnp.zeros((nb_, 1), _F32)
        ts_ref[...] = jnp.concatenate(
            [jnp.dot(hn, ws_ref[...], preferred_element_type=_F32), zcol, pad],
            axis=1)
        td_ref[...] = jnp.concatenate(
            [jnp.dot(hn, wd_ref[...], preferred_element_type=_F32), par, pad],
            axis=1)

    return pl.pallas_call(
        body,
        grid=(_N // nb,),
        in_specs=[pl.BlockSpec((nb, _D), lambda i: (i, 0)),
                  pl.BlockSpec((nb, _D), lambda i: (i, 0)),
                  pl.BlockSpec((nb, _D), lambda i: (i, 0)),
                  _full(w1h), _full(w1a), _full(b1), _full(w2), _full(b2),
                  _full(wns), _full(wnd)],
        out_specs=[pl.BlockSpec((nb, _D), lambda i: (i, 0)),
                   pl.BlockSpec((nb, 128), lambda i: (i, 0)),
                   pl.BlockSpec((nb, 128), lambda i: (i, 0))],
        out_shape=[jax.ShapeDtypeStruct((_N, _D), _F32),
                   jax.ShapeDtypeStruct((_N, 128), _F32),
                   jax.ShapeDtypeStruct((_N, 128), _F32)],
    )(h, a0, a1, w1h, w1a, b1, w2, b2, wns, wnd)


def _node_final(h, a0, a1, aux, w1h, w1a, b1, w2, b2, wd1, bd1, wd2, bd2):
    nb = _NB

    def body(h_ref, a0_ref, a1_ref, aux_ref, w1h_ref, w1a_ref, b1_ref,
             w2_ref, b2_ref, wd1_ref, bd1_ref, wd2_ref, bd2_ref,
             nxt_ref, loss_ref):
        h_ = h_ref[...]
        agg = a0_ref[...] + a1_ref[...]
        z = jax.nn.relu(jnp.dot(h_, w1h_ref[...], preferred_element_type=_F32)
                        + jnp.dot(agg, w1a_ref[...], preferred_element_type=_F32)
                        + b1_ref[...])
        hn = h_ + jnp.dot(z, w2_ref[...], preferred_element_type=_F32) \
            + b2_ref[...]
        za = jax.nn.relu(jnp.dot(hn, wd1_ref[...], preferred_element_type=_F32)
                         + bd1_ref[...])
        accel = jnp.dot(za, wd2_ref[...], preferred_element_type=_F32) \
            + bd2_ref[...]
        aux_ = aux_ref[...]
        last = aux_[:, 0:3]
        prev = aux_[:, 3:6]
        gt = aux_[:, 6:9]
        pt = aux_[:, 9:10]
        nxt = last + (last - prev) + accel
        nxt = jnp.where(pt == 3.0, gt, nxt)
        nxt_ref[...] = nxt
        df = nxt - gt

        @pl.when(pl.program_id(0) == 0)
        def _():
            loss_ref[0, 0] = 0.0

        loss_ref[0, 0] += jnp.sum(df * df)

    return pl.pallas_call(
        body,
        grid=(_N // nb,),
        in_specs=[pl.BlockSpec((nb, _D), lambda i: (i, 0)),
                  pl.BlockSpec((nb, _D), lambda i: (i, 0)),
                  pl.BlockSpec((nb, _D), lambda i: (i, 0)),
                  pl.BlockSpec((nb, 10), lambda i: (i, 0)),
                  _full(w1h), _full(w1a), _full(b1), _full(w2), _full(b2),
                  _full(wd1), _full(bd1), _full(wd2), _full(bd2)],
        out_specs=[pl.BlockSpec((nb, 3), lambda i: (i, 0)),
                   pl.BlockSpec((1, 1), lambda i: (0, 0),
                                memory_space=pltpu.SMEM)],
        out_shape=[jax.ShapeDtypeStruct((_N, 3), _F32),
                   jax.ShapeDtypeStruct((1, 1), _F32)],
    )(h, a0, a1, aux, w1h, w1a, b1, w2, b2, wd1, bd1, wd2, bd2)


def _make_gather(width):
    mesh = plsc.VectorSubcoreMesh(core_axis_name="c", subcore_axis_name="s")

    @functools.partial(
        pl.kernel,
        out_type=jax.ShapeDtypeStruct((_E, width), _F32),
        mesh=mesh,
        scratch_types=[
            pltpu.VMEM((_CHUNK,), jnp.int32),
            pltpu.VMEM((_CHUNK,), jnp.int32),
            pltpu.VMEM((_CHUNK,), jnp.int32),
            pltpu.VMEM((_CHUNK,), jnp.int32),
            pltpu.VMEM((_CHUNK, width), _F32),
            pltpu.VMEM((_CHUNK, width), _F32),
            pltpu.VMEM((_CHUNK, width), _F32),
            pltpu.VMEM((_CHUNK, width), _F32),
            pltpu.SemaphoreType.DMA,
            pltpu.SemaphoreType.DMA,
            pltpu.SemaphoreType.DMA,
            pltpu.SemaphoreType.DMA,
        ],
    )
    def gather_kernel(ts, td, src, dst, out, sidx0, didx0, sidx1, didx1,
                      bufa0, bufb0, bufa1, bufb1, sa0, sb0, sa1, sb1):
        cid = lax.axis_index("c")
        sid = lax.axis_index("s")
        base = (cid * 16 + sid) * _PERW
        slots = ((sidx0, didx0, bufa0, bufb0, sa0, sb0),
                 (sidx1, didx1, bufa1, bufb1, sa1, sb1))

        def issue(j, b):
            off = base + j * _CHUNK
            si, di, ba, bb, sas, sbs = slots[b]
            pltpu.sync_copy(src.at[pl.ds(off, _CHUNK)], si)
            pltpu.sync_copy(dst.at[pl.ds(off, _CHUNK)], di)
            pltpu.async_copy(ts.at[si], ba, sas)
            pltpu.async_copy(td.at[di], bb, sbs)

        def drain_and_emit(j, b):
            off = base + j * _CHUNK
            si, di, ba, bb, sas, sbs = slots[b]
            pltpu.make_async_copy(ts.at[si], ba, sas).wait()
            pltpu.make_async_copy(td.at[di], bb, sbs).wait()

            def row_body(i, carry2):
                for w in range(width // 16):
                    sl = pl.ds(w * 16, 16)
                    ba[i, sl] = ba[i, sl] + bb[i, sl]
                return carry2

            lax.fori_loop(0, _CHUNK, row_body, 0)
            pltpu.sync_copy(ba, out.at[pl.ds(off, _CHUNK)])

        issue(0, 0)

        def pair_body(j2, carry):
            j = j2 * 2
            issue(j + 1, 1)
            drain_and_emit(j, 0)

            @pl.when(j + 2 < _NCHUNKS)
            def _():
                issue(j + 2, 0)

            drain_and_emit(j + 1, 1)
            return carry

        lax.fori_loop(0, _NCHUNKS // 2, pair_body, 0)

    return gather_kernel


def _make_scatter():
    mesh = plsc.VectorSubcoreMesh(core_axis_name="c", subcore_axis_name="s")
    half = _HALFP
    rpt = half // 10

    @functools.partial(
        pl.kernel,
        out_type=jax.ShapeDtypeStruct((2 * half, 128), _F32),
        mesh=mesh,
        scratch_types=[
            pltpu.VMEM((_CHUNK,), jnp.int32),
            pltpu.VMEM((_CHUNK,), jnp.int32),
            pltpu.VMEM((_CHUNK, 128), _F32),
            pltpu.VMEM((_CHUNK, 128), _F32),
            pltpu.VMEM_SHARED((half, 128), _F32),
            pltpu.SemaphoreType.DMA,
            pltpu.SemaphoreType.DMA,
        ],
    )
    def scatter_kernel(m, dstv, zeros, out, didx0, didx1, mbuf0, mbuf1,
                       shared, sm0, sm1):
        cid = lax.axis_index("c")
        sid = lax.axis_index("s")

        @pl.when(sid == 0)
        def _():
            pltpu.sync_copy(zeros, shared)

        plsc.subcore_barrier()
        base = (cid * 16 + sid) * _PERW
        slots = ((didx0, mbuf0, sm0), (didx1, mbuf1, sm1))

        def issue(j, b):
            off = base + j * _CHUNK
            di, mb, sm = slots[b]
            pltpu.sync_copy(dstv.at[pl.ds(off, _CHUNK)], di)
            pltpu.async_copy(m.at[pl.ds(off, _CHUNK)], mb, sm)

        def drain_and_add(j, b):
            off = base + j * _CHUNK
            di, mb, sm = slots[b]
            pltpu.make_async_copy(m.at[pl.ds(off, _CHUNK)], mb, sm).wait()
            pltpu.sync_copy(mb, shared.at[di], add=True)

        issue(0, 0)

        def pair_body(j2, carry):
            j = j2 * 2
            issue(j + 1, 1)
            drain_and_add(j, 0)

            @pl.when(j + 2 < _NCHUNKS)
            def _():
                issue(j + 2, 0)

            drain_and_add(j + 1, 1)
            return carry

        lax.fori_loop(0, _NCHUNKS // 2, pair_body, 0)
        plsc.subcore_barrier()

        @pl.when(sid < 10)
        def _():
            r0 = sid * rpt
            pltpu.sync_copy(shared.at[pl.ds(r0, rpt)],
                            out.at[pl.ds(cid * half + r0, rpt)])

    return scatter_kernel


_gather128 = _make_gather(128)
_scatter = _make_scatter()


def kernel(position, n_particles_per_example, particle_type, step_context,
           edge_index, ct0, ct1, ct2, type_embedding, node_enc, edge_enc,
           mp1_edge, mp1_node, mp2_edge, mp2_node, decoder):
    del n_particles_per_example, step_context
    (w1n, b1n), (w2n, b2n) = node_enc
    (w1e, b1e), (w2e, b2e) = edge_enc
    (w1m1, b1m1), (w2m1, b2m1) = mp1_edge
    (w1u1, b1u1), (w2u1, b2u1) = mp1_node
    (w1m2, b1m2), (w2m2, b2m2) = mp2_edge
    (w1u2, b1u2), (w2u2, b2u2) = mp2_node
    (wd1, bd1), (wd2, bd2) = decoder

    gctx = jnp.concatenate([(ct0 * 2900.0 + 100.0).reshape(1),
                            (ct1 * 195.0 + 5.0).reshape(1),
                            (ct2 * 0.45).reshape(1)])
    te9 = type_embedding @ w1n[15:31]
    gconst = (gctx @ w1n[31:34] + b1n).reshape(1, _D)
    w1nv = w1n[0:15]

    ptype_f = particle_type.astype(_F32)[:, None]
    src = edge_index[0]
    dst = edge_index[1]
    dhalf = dst // 2
    cur = position[:, 0:_TIN]
    gt = position[:, _TIN:_TIN + _STEPS]
    zeros = jnp.zeros((_HALFP, 128), _F32)

    preds = []
    losses = []
    for step in range(_STEPS):
        nodein = jnp.concatenate([cur.reshape(_N, 18), ptype_f], axis=1)
        h, ts, td = _node_prep(nodein, w1nv, te9, gconst, w2n, _vec(b2n),
                               w1m1[64:128], w1m1[128:192])
        g80 = _gather128(ts, td, src, dst)
        e1, m1 = _edge_round1(g80, w1e, _vec(b1e), w2e, _vec(b2e),
                              w1m1[0:64], _vec(b1m1), w2m1, _vec(b2m1))
        agg1 = _scatter(m1, dhalf, zeros)
        a0 = agg1[:_N // 2].reshape(_N, _D)
        a1 = agg1[_HALFP:_HALFP + _N // 2].reshape(_N, _D)
        h, ts2, td2 = _node_update(h, a0, a1, w1u1[0:64], w1u1[64:128],
                                   _vec(b1u1), w2u1, _vec(b2u1),
                                   w1m2[64:128], w1m2[128:192])
        g64 = _gather128(ts2, td2, src, dst)
        m2 = _edge_round2(e1, g64, w1m2[0:64], _vec(b1m2), w2m2, _vec(b2m2))
        agg2 = _scatter(m2, dhalf, zeros)
        b0 = agg2[:_N // 2].reshape(_N, _D)
        b1_ = agg2[_HALFP:_HALFP + _N // 2].reshape(_N, _D)
        aux = jnp.concatenate([cur[:, -1], cur[:, -2], gt[:, step], ptype_f],
                              axis=1)
        nxt, lstep = _node_final(h, b0, b1_, aux, w1u2[0:64], w1u2[64:128],
                                 _vec(b1u2), w2u2, _vec(b2u2), wd1, _vec(bd1),
                                 wd2, _vec(bd2))
        preds.append(nxt)
        losses.append(lstep[0, 0])
        cur = jnp.concatenate([cur[:, 1:], nxt[:, None, :]], axis=1)

    preds = jnp.stack(preds)
    gt_p = jnp.transpose(gt, (1, 0, 2))
    loss = losses[0] + losses[1] + losses[2]
    return (loss, preds, gt_p)
